# Initial kernel scaffold; baseline (speedup 1.0000x reference)
import functools

import jax
import jax.numpy as jnp
from jax import lax
from jax.experimental import pallas as pl
from jax.experimental.pallas import tpu as pltpu
from jax.experimental.pallas import tpu_sc as plsc

VOCAB = 1000000
DIM = 32
B = 4096
L = 200
N = B * L

_INFO = plsc.get_sparse_core_info()
NC = _INFO.num_cores
NS = _INFO.num_subcores
NW = NC * NS
B_PER_W = N // NW

CHUNK = 1024
SEG = 128
K = CHUNK // SEG
NCHUNK = B_PER_W // CHUNK


def _make_sc_gather():
  mesh = plsc.VectorSubcoreMesh(core_axis_name="c", subcore_axis_name="s")

  @functools.partial(
      pl.kernel,
      mesh=mesh,
      out_type=jax.ShapeDtypeStruct((N, DIM), jnp.float32),
      scratch_types=[
          pltpu.VMEM((CHUNK,), jnp.int32),
          pltpu.VMEM((CHUNK, DIM), jnp.float32),
          pltpu.SemaphoreType.DMA,
      ],
  )
  def gather_kernel(idx_hbm, table_hbm, out_hbm, idx_v, rows_v, sem):
    wid = lax.axis_index("s") * NC + lax.axis_index("c")
    base = wid * B_PER_W

    def chunk_body(g, carry):
      off = base + g * CHUNK
      pltpu.sync_copy(idx_hbm.at[pl.ds(off, CHUNK)], idx_v)
      copies = []
      for j in range(K):
        copies.append(
            pltpu.async_copy(
                table_hbm.at[idx_v.at[pl.ds(j * SEG, SEG)]],
                rows_v.at[pl.ds(j * SEG, SEG)],
                sem,
            ))
      for c in copies:
        c.wait()
      pltpu.sync_copy(rows_v, out_hbm.at[pl.ds(off, CHUNK)])
      return carry

    lax.fori_loop(0, NCHUNK, chunk_body, 0)

  return gather_kernel


_sc_gather = _make_sc_gather()


@jax.jit
def kernel(inputs, table):
  idx = inputs.reshape(N).astype(jnp.int32)
  out = _sc_gather(idx, table)
  return out.reshape(B, L, DIM)


# speedup vs baseline: 1.4580x; 1.4580x over previous
import functools

import jax
import jax.numpy as jnp
from jax import lax
from jax.experimental import pallas as pl
from jax.experimental.pallas import tpu as pltpu
from jax.experimental.pallas import tpu_sc as plsc

VOCAB = 1000000
DIM = 32
B = 4096
L = 200
N = B * L

_INFO = plsc.get_sparse_core_info()
NC = _INFO.num_cores
NS = _INFO.num_subcores
NW = NC * NS
B_PER_W = N // NW

CHUNK = 1024
SEG = 128
K = CHUNK // SEG
NCHUNK = B_PER_W // CHUNK


def _make_sc_gather():
  mesh = plsc.VectorSubcoreMesh(core_axis_name="c", subcore_axis_name="s")

  @functools.partial(
      pl.kernel,
      mesh=mesh,
      compiler_params=pltpu.CompilerParams(use_tc_tiling_on_sc=False),
      out_type=jax.ShapeDtypeStruct((N, DIM), jnp.float32),
      scratch_types=[
          pltpu.VMEM((CHUNK,), jnp.int32),
          pltpu.VMEM((CHUNK, DIM), jnp.float32),
          pltpu.SemaphoreType.DMA,
      ],
  )
  def gather_kernel(idx_hbm, table_hbm, out_hbm, idx_v, rows_v, sem):
    wid = lax.axis_index("s") * NC + lax.axis_index("c")
    base = wid * B_PER_W

    def chunk_body(g, carry):
      off = base + g * CHUNK
      pltpu.sync_copy(idx_hbm.at[pl.ds(off, CHUNK)], idx_v)
      copies = []
      for j in range(K):
        copies.append(
            pltpu.async_copy(
                table_hbm.at[idx_v.at[pl.ds(j * SEG, SEG)]],
                rows_v.at[pl.ds(j * SEG, SEG)],
                sem,
            ))
      for c in copies:
        c.wait()
      pltpu.sync_copy(rows_v, out_hbm.at[pl.ds(off, CHUNK)])
      return carry

    lax.fori_loop(0, NCHUNK, chunk_body, 0)

  return gather_kernel


_sc_gather = _make_sc_gather()


@jax.jit
def kernel(inputs, table):
  idx = inputs.reshape(N).astype(jnp.int32)
  out = _sc_gather(idx, table)
  return out.reshape(B, L, DIM)


# speedup vs baseline: 1.4872x; 1.0201x over previous
import functools

import jax
import jax.numpy as jnp
from jax import lax
from jax.experimental import pallas as pl
from jax.experimental.pallas import tpu as pltpu
from jax.experimental.pallas import tpu_sc as plsc

VOCAB = 1000000
DIM = 32
B = 4096
L = 200
N = B * L

_INFO = plsc.get_sparse_core_info()
NC = _INFO.num_cores
NS = _INFO.num_subcores
NW = NC * NS
B_PER_W = N // NW

CHUNK = 1280
SEG = 128
K = CHUNK // SEG
NCHUNK = B_PER_W // CHUNK
NBUF = 2


def _make_sc_gather():
  mesh = plsc.VectorSubcoreMesh(core_axis_name="c", subcore_axis_name="s")

  @functools.partial(
      pl.kernel,
      mesh=mesh,
      compiler_params=pltpu.CompilerParams(use_tc_tiling_on_sc=False),
      out_type=jax.ShapeDtypeStruct((N, DIM), jnp.float32),
      scratch_types=[
          pltpu.VMEM((NBUF, CHUNK), jnp.int32),
          pltpu.VMEM((NBUF, CHUNK, DIM), jnp.float32),
          pltpu.SemaphoreType.DMA((NBUF,)),
          pltpu.SemaphoreType.DMA((NBUF,)),
      ],
  )
  def gather_kernel(idx_hbm, table_hbm, out_hbm, idx_v, rows_v, gsem, osem):
    wid = lax.axis_index("s") * NC + lax.axis_index("c")
    base = wid * B_PER_W

    def fire_gathers(g, b):
      off = base + g * CHUNK
      pltpu.sync_copy(idx_hbm.at[pl.ds(off, CHUNK)], idx_v.at[b])
      for j in range(K):
        pltpu.async_copy(
            table_hbm.at[idx_v.at[b].at[pl.ds(j * SEG, SEG)]],
            rows_v.at[b].at[pl.ds(j * SEG, SEG)],
            gsem.at[b],
        )

    def wait_gathers(b):
      pltpu.make_async_copy(
          table_hbm.at[idx_v.at[b]], rows_v.at[b], gsem.at[b]).wait()

    def fire_out(g, b):
      off = base + g * CHUNK
      pltpu.async_copy(rows_v.at[b], out_hbm.at[pl.ds(off, CHUNK)], osem.at[b])

    def wait_out(g, b):
      off = base + g * CHUNK
      pltpu.make_async_copy(
          rows_v.at[b], out_hbm.at[pl.ds(off, CHUNK)], osem.at[b]).wait()

    for b in range(NBUF):
      fire_gathers(b, b)

    def body(g0, carry):
      for b in range(NBUF):
        g = g0 + b
        wait_gathers(b)
        fire_out(g, b)
        ng = g + NBUF

        @pl.when(ng < NCHUNK)
        def _():
          wait_out(g, b)
          fire_gathers(ng, b)

        @pl.when(ng >= NCHUNK)
        def _():
          wait_out(g, b)
      return carry

    lax.fori_loop(0, NCHUNK // NBUF, lambda i, c: body(i * NBUF, c), 0,
                  unroll=False)

  return gather_kernel


_sc_gather = _make_sc_gather()


@jax.jit
def kernel(inputs, table):
  idx = inputs.reshape(N).astype(jnp.int32)
  out = _sc_gather(idx, table)
  return out.reshape(B, L, DIM)


# speedup vs baseline: 1.5014x; 1.0095x over previous
import functools

import jax
import jax.numpy as jnp
from jax import lax
from jax.experimental import pallas as pl
from jax.experimental.pallas import tpu as pltpu
from jax.experimental.pallas import tpu_sc as plsc

VOCAB = 1000000
DIM = 32
B = 4096
L = 200

_INFO = plsc.get_sparse_core_info()
NC = _INFO.num_cores
NS = _INFO.num_subcores
NW = NC * NS
BW = B // NW

CB = 4
SEGS = ((0, 104), (104, 96))
NCHUNK = BW // CB
NBUF = 2


def _make_sc_gather():
  mesh = plsc.VectorSubcoreMesh(core_axis_name="c", subcore_axis_name="s")

  @functools.partial(
      pl.kernel,
      mesh=mesh,
      compiler_params=pltpu.CompilerParams(use_tc_tiling_on_sc=False),
      out_type=jax.ShapeDtypeStruct((B, L, DIM), jnp.float32),
      scratch_types=[
          pltpu.VMEM((BW, L), jnp.int32),
          pltpu.VMEM((NBUF, CB, L, DIM), jnp.float32),
          pltpu.SemaphoreType.DMA((NBUF,)),
          pltpu.SemaphoreType.DMA((NBUF,)),
      ],
  )
  def gather_kernel(idx_hbm, table_hbm, out_hbm, idx_v, rows_v, gsem, osem):
    wid = lax.axis_index("s") * NC + lax.axis_index("c")
    b_base = wid * BW

    pltpu.sync_copy(idx_hbm.at[pl.ds(b_base, BW)], idx_v)

    def each_stream(g, slot, fn):
      for r in range(CB):
        for off, seg in SEGS:
          idx_slice = idx_v.at[g * CB + r].at[pl.ds(off, seg)]
          dst = rows_v.at[slot].at[r].at[pl.ds(off, seg)]
          fn(table_hbm.at[idx_slice], dst, gsem.at[slot])

    def fire_gathers(g, slot):
      each_stream(g, slot, pltpu.async_copy)

    def wait_gathers(g, slot):
      each_stream(g, slot, lambda s, d, m: pltpu.make_async_copy(s, d, m).wait())

    def fire_out(g, slot):
      pltpu.async_copy(rows_v.at[slot],
                       out_hbm.at[pl.ds(b_base + g * CB, CB)], osem.at[slot])

    def wait_out(g, slot):
      pltpu.make_async_copy(rows_v.at[slot],
                            out_hbm.at[pl.ds(b_base + g * CB, CB)],
                            osem.at[slot]).wait()

    for b in range(NBUF):
      fire_gathers(b, b)

    def body(g0, carry):
      for b in range(NBUF):
        g = g0 + b
        wait_gathers(g, b)
        fire_out(g, b)
        ng = g + NBUF

        @pl.when(ng < NCHUNK)
        def _():
          wait_out(g, b)
          fire_gathers(ng, b)

        @pl.when(ng >= NCHUNK)
        def _():
          wait_out(g, b)
      return carry

    lax.fori_loop(0, NCHUNK // NBUF, lambda i, c: body(i * NBUF, c), 0,
                  unroll=False)

  return gather_kernel


_sc_gather = _make_sc_gather()


@jax.jit
def kernel(inputs, table):
  return _sc_gather(inputs.astype(jnp.int32), table)
